# Initial kernel scaffold; baseline (speedup 1.0000x reference)
import jax
import jax.numpy as jnp
from jax.experimental import pallas as pl


def kernel(boxes, scores, idxs):
    raise NotImplementedError("write your pallas kernel here")


# speedup vs baseline: 24.7111x; 24.7111x over previous
import jax
import jax.numpy as jnp
from jax.experimental import pallas as pl
from jax.experimental.pallas import tpu as pltpu

N = 5000
ROWS = 40
LANES = 128
PAD = ROWS * LANES
IOU_THRESHOLD = 0.5
SCORE_THRESHOLD = 0.05
NEG_INF = float("-inf")


def _nms_loop_kernel(x1_ref, y1_ref, x2_ref, y2_ref, cls_ref, s_ref,
                     order_ref, score_ref):
    x1 = x1_ref[...]
    y1 = y1_ref[...]
    x2 = x2_ref[...]
    y2 = y2_ref[...]

    max_coord = jnp.max(jnp.maximum(jnp.maximum(x1, y1), jnp.maximum(x2, y2)))
    off = cls_ref[...] * (max_coord + 1.0)
    x1 = x1 + off
    y1 = y1 + off
    x2 = x2 + off
    y2 = y2 + off
    area = (x2 - x1) * (y2 - y1)

    idx = (jax.lax.broadcasted_iota(jnp.int32, (ROWS, LANES), 0) * LANES
           + jax.lax.broadcasted_iota(jnp.int32, (ROWS, LANES), 1))
    big = jnp.int32(2**31 - 1)

    def body(i, s):
        m_score = jnp.max(s)
        m = jnp.min(jnp.where(s == m_score, idx, big))
        sel = idx == m
        bx1 = jnp.max(jnp.where(sel, x1, NEG_INF))
        by1 = jnp.max(jnp.where(sel, y1, NEG_INF))
        bx2 = jnp.max(jnp.where(sel, x2, NEG_INF))
        by2 = jnp.max(jnp.where(sel, y2, NEG_INF))
        area_m = (bx2 - bx1) * (by2 - by1)
        w = jnp.maximum(jnp.minimum(bx2, x2) - jnp.maximum(bx1, x1), 0.0)
        h = jnp.maximum(jnp.minimum(by2, y2) - jnp.maximum(by1, y1), 0.0)
        inter = w * h
        iou = inter / (area_m + area - inter + 1e-12)
        decay = jnp.where(iou > IOU_THRESHOLD, 1.0 - iou, 1.0)
        others = (s != NEG_INF) & jnp.logical_not(sel)
        s = jnp.where(others, s * decay, s)
        s = jnp.where(sel, NEG_INF, s)
        order_ref[i] = m
        score_ref[i] = m_score
        return s

    jax.lax.fori_loop(0, N, body, s_ref[...])


def kernel(boxes, scores, idxs):
    pad = PAD - N
    x1 = jnp.pad(boxes[:, 0], (0, pad)).reshape(ROWS, LANES)
    y1 = jnp.pad(boxes[:, 1], (0, pad)).reshape(ROWS, LANES)
    x2 = jnp.pad(boxes[:, 2], (0, pad)).reshape(ROWS, LANES)
    y2 = jnp.pad(boxes[:, 3], (0, pad)).reshape(ROWS, LANES)
    cls = jnp.pad(idxs.astype(jnp.float32), (0, pad)).reshape(ROWS, LANES)
    s = jnp.pad(scores, (0, pad), constant_values=NEG_INF).reshape(ROWS, LANES)

    order, sel_scores = pl.pallas_call(
        _nms_loop_kernel,
        in_specs=[pl.BlockSpec(memory_space=pltpu.VMEM)] * 6,
        out_specs=(pl.BlockSpec(memory_space=pltpu.SMEM),
                   pl.BlockSpec(memory_space=pltpu.SMEM)),
        out_shape=(jax.ShapeDtypeStruct((N,), jnp.int32),
                   jax.ShapeDtypeStruct((N,), jnp.float32)),
    )(x1, y1, x2, y2, cls, s)

    keep_mask = sel_scores > SCORE_THRESHOLD
    return order, sel_scores, keep_mask


# speedup vs baseline: 32.9199x; 1.3322x over previous
import jax
import jax.numpy as jnp
from jax.experimental import pallas as pl
from jax.experimental.pallas import tpu as pltpu

N = 5000
ROWS = 40
LANES = 128
PAD = ROWS * LANES
IOU_THRESHOLD = 0.5
SCORE_THRESHOLD = 0.05
NEG_INF = float("-inf")


def _nms_loop_kernel(x1_ref, y1_ref, x2_ref, y2_ref, s_ref,
                     x1s_ref, y1s_ref, x2s_ref, y2s_ref,
                     order_ref, score_ref):
    x1 = x1_ref[...]
    y1 = y1_ref[...]
    x2 = x2_ref[...]
    y2 = y2_ref[...]
    area = (x2 - x1) * (y2 - y1)

    idx = (jax.lax.broadcasted_iota(jnp.int32, (ROWS, LANES), 0) * LANES
           + jax.lax.broadcasted_iota(jnp.int32, (ROWS, LANES), 1))
    big = jnp.int32(2**31 - 1)

    def body(i, s):
        m_score = jnp.max(s)
        m = jnp.min(jnp.where(s == m_score, idx, big))
        bx1 = x1s_ref[m]
        by1 = y1s_ref[m]
        bx2 = x2s_ref[m]
        by2 = y2s_ref[m]
        area_m = (bx2 - bx1) * (by2 - by1)
        w = jnp.maximum(jnp.minimum(bx2, x2) - jnp.maximum(bx1, x1), 0.0)
        h = jnp.maximum(jnp.minimum(by2, y2) - jnp.maximum(by1, y1), 0.0)
        inter = w * h
        iou = inter / (area_m + area - inter + 1e-12)
        decay = jnp.where(iou > IOU_THRESHOLD, 1.0 - iou, 1.0)
        others = (s != NEG_INF) & (idx != m)
        s = jnp.where(others, s * decay, s)
        s = jnp.where(idx == m, NEG_INF, s)
        order_ref[i] = m
        score_ref[i] = m_score
        return s

    jax.lax.fori_loop(0, N, body, s_ref[...])


def kernel(boxes, scores, idxs):
    max_coord = jnp.max(boxes)
    offsets = idxs.astype(boxes.dtype) * (max_coord + 1.0)
    obox = boxes + offsets[:, None]

    pad = PAD - N
    x1 = jnp.pad(obox[:, 0], (0, pad)).reshape(ROWS, LANES)
    y1 = jnp.pad(obox[:, 1], (0, pad)).reshape(ROWS, LANES)
    x2 = jnp.pad(obox[:, 2], (0, pad)).reshape(ROWS, LANES)
    y2 = jnp.pad(obox[:, 3], (0, pad)).reshape(ROWS, LANES)
    s = jnp.pad(scores, (0, pad), constant_values=NEG_INF).reshape(ROWS, LANES)

    vspec = pl.BlockSpec(memory_space=pltpu.VMEM)
    sspec = pl.BlockSpec(memory_space=pltpu.SMEM)
    order, sel_scores = pl.pallas_call(
        _nms_loop_kernel,
        in_specs=[vspec] * 5 + [sspec] * 4,
        out_specs=(sspec, sspec),
        out_shape=(jax.ShapeDtypeStruct((N,), jnp.int32),
                   jax.ShapeDtypeStruct((N,), jnp.float32)),
    )(x1, y1, x2, y2, s, obox[:, 0], obox[:, 1], obox[:, 2], obox[:, 3])

    keep_mask = sel_scores > SCORE_THRESHOLD
    return order, sel_scores, keep_mask


# speedup vs baseline: 42.2978x; 1.2849x over previous
import jax
import jax.numpy as jnp
from jax.experimental import pallas as pl
from jax.experimental.pallas import tpu as pltpu

N = 5000
ROWS = 40
LANES = 128
PAD = ROWS * LANES
IOU_THRESHOLD = 0.5
SCORE_THRESHOLD = 0.05
NEG_INF = float("-inf")


def _nms_loop_kernel(x1_ref, y1_ref, x2_ref, y2_ref, s_ref,
                     x1s_ref, y1s_ref, x2s_ref, y2s_ref,
                     order_ref, score_ref):
    x1 = x1_ref[...]
    y1 = y1_ref[...]
    x2 = x2_ref[...]
    y2 = y2_ref[...]
    area = (x2 - x1) * (y2 - y1)

    idx = (jax.lax.broadcasted_iota(jnp.int32, (ROWS, LANES), 0) * LANES
           + jax.lax.broadcasted_iota(jnp.int32, (ROWS, LANES), 1))
    idx_f = idx.astype(jnp.float32)
    BIG_F = jnp.float32(3.0e38)

    def body(i, s):
        sf = jnp.max(s.reshape(5, 8, LANES), axis=0)
        sf = jnp.max(sf, axis=0, keepdims=True)
        m_score = jnp.max(sf)
        kf = jnp.min(jnp.where(s == m_score, idx_f, BIG_F).reshape(
            5, 8, LANES), axis=0)
        kf = jnp.min(kf, axis=0, keepdims=True)
        m = jnp.min(kf).astype(jnp.int32)
        bx1 = x1s_ref[m]
        by1 = y1s_ref[m]
        bx2 = x2s_ref[m]
        by2 = y2s_ref[m]
        area_m = (bx2 - bx1) * (by2 - by1)
        w = jnp.maximum(jnp.minimum(bx2, x2) - jnp.maximum(bx1, x1), 0.0)
        h = jnp.maximum(jnp.minimum(by2, y2) - jnp.maximum(by1, y1), 0.0)
        inter = w * h
        iou = inter / (area_m + area - inter + 1e-12)
        decay = jnp.where(iou > IOU_THRESHOLD, 1.0 - iou, 1.0)
        others = (s != NEG_INF) & (idx != m)
        s = jnp.where(others, s * decay, s)
        s = jnp.where(idx == m, NEG_INF, s)
        order_ref[i] = m
        score_ref[i] = m_score
        return s

    jax.lax.fori_loop(0, N, body, s_ref[...])


def kernel(boxes, scores, idxs):
    max_coord = jnp.max(boxes)
    offsets = idxs.astype(boxes.dtype) * (max_coord + 1.0)
    obox = boxes + offsets[:, None]

    pad = PAD - N
    x1 = jnp.pad(obox[:, 0], (0, pad)).reshape(ROWS, LANES)
    y1 = jnp.pad(obox[:, 1], (0, pad)).reshape(ROWS, LANES)
    x2 = jnp.pad(obox[:, 2], (0, pad)).reshape(ROWS, LANES)
    y2 = jnp.pad(obox[:, 3], (0, pad)).reshape(ROWS, LANES)
    s = jnp.pad(scores, (0, pad), constant_values=NEG_INF).reshape(ROWS, LANES)

    vspec = pl.BlockSpec(memory_space=pltpu.VMEM)
    sspec = pl.BlockSpec(memory_space=pltpu.SMEM)
    order, sel_scores = pl.pallas_call(
        _nms_loop_kernel,
        in_specs=[vspec] * 5 + [sspec] * 4,
        out_specs=(sspec, sspec),
        out_shape=(jax.ShapeDtypeStruct((N,), jnp.int32),
                   jax.ShapeDtypeStruct((N,), jnp.float32)),
    )(x1, y1, x2, y2, s, obox[:, 0], obox[:, 1], obox[:, 2], obox[:, 3])

    keep_mask = sel_scores > SCORE_THRESHOLD
    return order, sel_scores, keep_mask


# speedup vs baseline: 203.4876x; 4.8108x over previous
import functools

import jax
import jax.numpy as jnp
from jax.experimental import pallas as pl
from jax.experimental.pallas import tpu as pltpu

N = 5000
NUM_CLASSES = 80
CAP = 128
CROWS = NUM_CLASSES
SROWS = 128
LANES = 128
GROWS = 40
GPAD = GROWS * LANES
IOU_THRESHOLD = 0.5
SCORE_THRESHOLD = 0.05
NEG_INF = float("-inf")
BIG_G = float(2**24 - 1)


def _vec_kernel(x1_ref, y1_ref, x2_ref, y2_ref, s_ref, g_ref,
                score_out_ref, gidx_out_ref):
    x1 = x1_ref[...]
    y1 = y1_ref[...]
    x2 = x2_ref[...]
    y2 = y2_ref[...]
    area = (x2 - x1) * (y2 - y1)
    lane = jax.lax.broadcasted_iota(
        jnp.int32, (CROWS, LANES), 1).astype(jnp.float32)
    big_lane = jnp.float32(1e9)

    def body(_, carry):
        sw, sf = carry
        m = jnp.max(sw, axis=1, keepdims=True)
        lwin = jnp.min(jnp.where(sw == m, lane, big_lane), axis=1,
                       keepdims=True)
        onehot = lane == lwin
        bx1 = jnp.max(jnp.where(onehot, x1, NEG_INF), axis=1, keepdims=True)
        by1 = jnp.max(jnp.where(onehot, y1, NEG_INF), axis=1, keepdims=True)
        bx2 = jnp.max(jnp.where(onehot, x2, NEG_INF), axis=1, keepdims=True)
        by2 = jnp.max(jnp.where(onehot, y2, NEG_INF), axis=1, keepdims=True)
        area_m = (bx2 - bx1) * (by2 - by1)
        w = jnp.maximum(jnp.minimum(bx2, x2) - jnp.maximum(bx1, x1), 0.0)
        h = jnp.maximum(jnp.minimum(by2, y2) - jnp.maximum(by1, y1), 0.0)
        inter = w * h
        iou = inter / (area_m + area - inter + 1e-12)
        decay = jnp.where(iou > IOU_THRESHOLD, 1.0 - iou, 1.0)
        others = (sw != NEG_INF) & jnp.logical_not(onehot)
        sw = jnp.where(others, sw * decay, sw)
        sf = jnp.where(others, sf * decay, sf)
        sw = jnp.where(onehot, NEG_INF, sw)
        return sw, sf

    s0 = s_ref[...]
    _, sf = jax.lax.fori_loop(0, CAP, body, (s0, s0))

    S = jnp.concatenate(
        [sf, jnp.full((SROWS - CROWS, LANES), NEG_INF, jnp.float32)], axis=0)
    G = jnp.concatenate(
        [g_ref[...], jnp.full((SROWS - CROWS, LANES), BIG_G, jnp.float32)],
        axis=0)

    ri = jax.lax.broadcasted_iota(jnp.int32, (SROWS, LANES), 0)
    li = jax.lax.broadcasted_iota(jnp.int32, (SROWS, LANES), 1)
    gi = ri * LANES + li

    def partner(a, j, axis, size):
        sh = j // LANES if axis == 0 else j
        lo = pltpu.roll(a, size - sh, axis=axis)
        hi = pltpu.roll(a, sh, axis=axis)
        return jnp.where((gi & j) != 0, hi, lo)

    def stage(S, G, j, k, axis, size):
        Ps = partner(S, j, axis, size)
        Pg = partner(G, j, axis, size)
        p_less = (Ps > S) | ((Ps == S) & (Pg < G))
        s_less = (S > Ps) | ((S == Ps) & (G < Pg))
        holdmin = ((gi & j) != 0) == ((gi & k) != 0)
        take = (holdmin & p_less) | (jnp.logical_not(holdmin) & s_less)
        return jnp.where(take, Ps, S), jnp.where(take, Pg, G)

    for lvl in range(1, 15):
        k = 1 << lvl
        n_row = max(0, lvl - 7)
        n_lane = min(lvl, 7)

        def row_stage(t, sg, k=k):
            j = (k >> 1) >> t
            return stage(sg[0], sg[1], j, k, 0, SROWS)

        def lane_stage(t, sg, k=k, n_lane=n_lane):
            j = (1 << (n_lane - 1)) >> t
            return stage(sg[0], sg[1], j, k, 1, LANES)

        if n_row:
            S, G = jax.lax.fori_loop(0, n_row, row_stage, (S, G))
        S, G = jax.lax.fori_loop(0, n_lane, lane_stage, (S, G))

    score_out_ref[...] = S
    gidx_out_ref[...] = G


def _vec_path(obox, scores, idxs):
    pos_in_class = jnp.cumsum(
        jax.nn.one_hot(idxs, NUM_CLASSES, dtype=jnp.int32), axis=0,
    )[jnp.arange(N), idxs] - 1
    pos = jnp.minimum(pos_in_class, CAP - 1)

    def scatter(vals, fill):
        z = jnp.full((CROWS, LANES), fill, jnp.float32)
        return z.at[idxs, pos].set(vals)

    x1 = scatter(obox[:, 0], 0.0)
    y1 = scatter(obox[:, 1], 0.0)
    x2 = scatter(obox[:, 2], 0.0)
    y2 = scatter(obox[:, 3], 0.0)
    s = scatter(scores, NEG_INF)
    g = scatter(jnp.arange(N, dtype=jnp.float32), BIG_G)

    vspec = pl.BlockSpec(memory_space=pltpu.VMEM)
    S, G = pl.pallas_call(
        _vec_kernel,
        in_specs=[vspec] * 6,
        out_specs=(vspec, vspec),
        out_shape=(jax.ShapeDtypeStruct((SROWS, LANES), jnp.float32),
                   jax.ShapeDtypeStruct((SROWS, LANES), jnp.float32)),
    )(x1, y1, x2, y2, s, g)

    order = G.reshape(-1)[:N].astype(jnp.int32)
    sel_scores = S.reshape(-1)[:N]
    return order, sel_scores


def _global_kernel(x1_ref, y1_ref, x2_ref, y2_ref, s_ref,
                   x1s_ref, y1s_ref, x2s_ref, y2s_ref,
                   order_ref, score_ref):
    x1 = x1_ref[...]
    y1 = y1_ref[...]
    x2 = x2_ref[...]
    y2 = y2_ref[...]
    area = (x2 - x1) * (y2 - y1)

    idx = (jax.lax.broadcasted_iota(jnp.int32, (GROWS, LANES), 0) * LANES
           + jax.lax.broadcasted_iota(jnp.int32, (GROWS, LANES), 1))
    idx_f = idx.astype(jnp.float32)
    big_f = jnp.float32(3.0e38)

    def body(i, s):
        sf = jnp.max(s.reshape(5, 8, LANES), axis=0)
        sf = jnp.max(sf, axis=0, keepdims=True)
        m_score = jnp.max(sf)
        kf = jnp.min(jnp.where(s == m_score, idx_f, big_f).reshape(
            5, 8, LANES), axis=0)
        kf = jnp.min(kf, axis=0, keepdims=True)
        m = jnp.min(kf).astype(jnp.int32)
        bx1 = x1s_ref[m]
        by1 = y1s_ref[m]
        bx2 = x2s_ref[m]
        by2 = y2s_ref[m]
        area_m = (bx2 - bx1) * (by2 - by1)
        w = jnp.maximum(jnp.minimum(bx2, x2) - jnp.maximum(bx1, x1), 0.0)
        h = jnp.maximum(jnp.minimum(by2, y2) - jnp.maximum(by1, y1), 0.0)
        inter = w * h
        iou = inter / (area_m + area - inter + 1e-12)
        decay = jnp.where(iou > IOU_THRESHOLD, 1.0 - iou, 1.0)
        others = (s != NEG_INF) & (idx != m)
        s = jnp.where(others, s * decay, s)
        s = jnp.where(idx == m, NEG_INF, s)
        order_ref[i] = m
        score_ref[i] = m_score
        return s

    jax.lax.fori_loop(0, N, body, s_ref[...])


def _global_path(obox, scores, idxs):
    del idxs
    pad = GPAD - N
    x1 = jnp.pad(obox[:, 0], (0, pad)).reshape(GROWS, LANES)
    y1 = jnp.pad(obox[:, 1], (0, pad)).reshape(GROWS, LANES)
    x2 = jnp.pad(obox[:, 2], (0, pad)).reshape(GROWS, LANES)
    y2 = jnp.pad(obox[:, 3], (0, pad)).reshape(GROWS, LANES)
    s = jnp.pad(scores, (0, pad), constant_values=NEG_INF).reshape(
        GROWS, LANES)

    vspec = pl.BlockSpec(memory_space=pltpu.VMEM)
    sspec = pl.BlockSpec(memory_space=pltpu.SMEM)
    order, sel_scores = pl.pallas_call(
        _global_kernel,
        in_specs=[vspec] * 5 + [sspec] * 4,
        out_specs=(sspec, sspec),
        out_shape=(jax.ShapeDtypeStruct((N,), jnp.int32),
                   jax.ShapeDtypeStruct((N,), jnp.float32)),
    )(x1, y1, x2, y2, s, obox[:, 0], obox[:, 1], obox[:, 2], obox[:, 3])
    return order, sel_scores


def kernel(boxes, scores, idxs):
    max_coord = jnp.max(boxes)
    offsets = idxs.astype(boxes.dtype) * (max_coord + 1.0)
    obox = boxes + offsets[:, None]

    counts = jnp.sum(jax.nn.one_hot(idxs, NUM_CLASSES, dtype=jnp.int32),
                     axis=0)
    fits = jnp.max(counts) <= CAP
    order, sel_scores = jax.lax.cond(
        fits, _vec_path, _global_path, obox, scores, idxs)
    keep_mask = sel_scores > SCORE_THRESHOLD
    return order, sel_scores, keep_mask


# speedup vs baseline: 305.4066x; 1.5009x over previous
import functools

import jax
import jax.numpy as jnp
from jax.experimental import pallas as pl
from jax.experimental.pallas import tpu as pltpu

N = 5000
NUM_CLASSES = 80
CAP = 128
CROWS = NUM_CLASSES
SROWS = 128
LANES = 128
GROWS = 40
GPAD = GROWS * LANES
IOU_THRESHOLD = 0.5
SCORE_THRESHOLD = 0.05
NEG_INF = float("-inf")
BIG_G = float(2**24 - 1)


def _vec_kernel(nmax_ref, x1_ref, y1_ref, x2_ref, y2_ref, s_ref, g_ref,
                score_out_ref, gidx_out_ref):
    x1 = x1_ref[...]
    y1 = y1_ref[...]
    x2 = x2_ref[...]
    y2 = y2_ref[...]
    area = (x2 - x1) * (y2 - y1)
    lane = jax.lax.broadcasted_iota(
        jnp.int32, (CROWS, LANES), 1).astype(jnp.float32)
    big_lane = jnp.float32(1e9)

    def body(_, carry):
        sw, sf = carry
        m = jnp.max(sw, axis=1, keepdims=True)
        lwin = jnp.min(jnp.where(sw == m, lane, big_lane), axis=1,
                       keepdims=True)
        onehot = lane == lwin
        bx1 = jnp.max(jnp.where(onehot, x1, NEG_INF), axis=1, keepdims=True)
        by1 = jnp.max(jnp.where(onehot, y1, NEG_INF), axis=1, keepdims=True)
        bx2 = jnp.max(jnp.where(onehot, x2, NEG_INF), axis=1, keepdims=True)
        by2 = jnp.max(jnp.where(onehot, y2, NEG_INF), axis=1, keepdims=True)
        area_m = (bx2 - bx1) * (by2 - by1)
        w = jnp.maximum(jnp.minimum(bx2, x2) - jnp.maximum(bx1, x1), 0.0)
        h = jnp.maximum(jnp.minimum(by2, y2) - jnp.maximum(by1, y1), 0.0)
        inter = w * h
        iou = inter / (area_m + area - inter + 1e-12)
        decay = jnp.where(iou > IOU_THRESHOLD, 1.0 - iou, 1.0)
        others = (sw != NEG_INF) & jnp.logical_not(onehot)
        sw = jnp.where(others, sw * decay, sw)
        sf = jnp.where(others, sf * decay, sf)
        sw = jnp.where(onehot, NEG_INF, sw)
        return sw, sf

    s0 = s_ref[...]
    _, sf = jax.lax.fori_loop(0, nmax_ref[0], body, (s0, s0))

    S = jnp.concatenate(
        [sf, jnp.full((SROWS - CROWS, LANES), NEG_INF, jnp.float32)], axis=0)
    G = jnp.concatenate(
        [g_ref[...], jnp.full((SROWS - CROWS, LANES), BIG_G, jnp.float32)],
        axis=0)

    ri = jax.lax.broadcasted_iota(jnp.int32, (SROWS, LANES), 0)
    li = jax.lax.broadcasted_iota(jnp.int32, (SROWS, LANES), 1)
    gi = ri * LANES + li

    def partner(a, j, axis, size):
        sh = j // LANES if axis == 0 else j
        lo = pltpu.roll(a, size - sh, axis=axis)
        hi = pltpu.roll(a, sh, axis=axis)
        return jnp.where((gi & j) != 0, hi, lo)

    def stage(S, G, j, k, axis, size):
        Ps = partner(S, j, axis, size)
        Pg = partner(G, j, axis, size)
        p_less = (Ps > S) | ((Ps == S) & (Pg < G))
        s_less = (S > Ps) | ((S == Ps) & (G < Pg))
        holdmin = ((gi & j) != 0) == ((gi & k) != 0)
        take = (holdmin & p_less) | (jnp.logical_not(holdmin) & s_less)
        return jnp.where(take, Ps, S), jnp.where(take, Pg, G)

    for lvl in range(1, 15):
        k = 1 << lvl
        n_row = max(0, lvl - 7)
        n_lane = min(lvl, 7)

        def row_stage(t, sg, k=k):
            j = (k >> 1) >> t
            return stage(sg[0], sg[1], j, k, 0, SROWS)

        def lane_stage(t, sg, k=k, n_lane=n_lane):
            j = (1 << (n_lane - 1)) >> t
            return stage(sg[0], sg[1], j, k, 1, LANES)

        if n_row:
            S, G = jax.lax.fori_loop(0, n_row, row_stage, (S, G))
        S, G = jax.lax.fori_loop(0, n_lane, lane_stage, (S, G))

    score_out_ref[...] = S
    gidx_out_ref[...] = G


def _vec_path(obox, scores, idxs, nmax):
    i_n = jnp.arange(N, dtype=jnp.int32)
    ks = jax.lax.sort(idxs * 8192 + i_n)
    cls_s = ks >> 13
    idx_s = ks & 8191
    seg_start = jnp.concatenate(
        [jnp.ones((1,), jnp.bool_), cls_s[1:] != cls_s[:-1]])
    start = jax.lax.cummax(jnp.where(seg_start, i_n, 0))
    pos_s = jnp.minimum(i_n - start, CAP - 1)
    pos = jnp.zeros((N,), jnp.int32).at[idx_s].set(pos_s)

    payload = jnp.stack(
        [obox[:, 0], obox[:, 1], obox[:, 2], obox[:, 3], scores,
         i_n.astype(jnp.float32)], axis=1)
    fill = jnp.array([0.0, 0.0, 0.0, 0.0, NEG_INF, BIG_G], jnp.float32)
    grouped = jnp.broadcast_to(
        fill, (CROWS, LANES, 6)).at[idxs, pos].set(payload)
    gx = grouped.transpose(2, 0, 1)
    x1, y1, x2, y2, s, g = (gx[0], gx[1], gx[2], gx[3], gx[4], gx[5])

    vspec = pl.BlockSpec(memory_space=pltpu.VMEM)
    sspec = pl.BlockSpec(memory_space=pltpu.SMEM)
    S, G = pl.pallas_call(
        _vec_kernel,
        in_specs=[sspec] + [vspec] * 6,
        out_specs=(vspec, vspec),
        out_shape=(jax.ShapeDtypeStruct((SROWS, LANES), jnp.float32),
                   jax.ShapeDtypeStruct((SROWS, LANES), jnp.float32)),
    )(nmax.reshape(1), x1, y1, x2, y2, s, g)

    order = G.reshape(-1)[:N].astype(jnp.int32)
    sel_scores = S.reshape(-1)[:N]
    return order, sel_scores


def _global_kernel(x1_ref, y1_ref, x2_ref, y2_ref, s_ref,
                   x1s_ref, y1s_ref, x2s_ref, y2s_ref,
                   order_ref, score_ref):
    x1 = x1_ref[...]
    y1 = y1_ref[...]
    x2 = x2_ref[...]
    y2 = y2_ref[...]
    area = (x2 - x1) * (y2 - y1)

    idx = (jax.lax.broadcasted_iota(jnp.int32, (GROWS, LANES), 0) * LANES
           + jax.lax.broadcasted_iota(jnp.int32, (GROWS, LANES), 1))
    idx_f = idx.astype(jnp.float32)
    big_f = jnp.float32(3.0e38)

    def body(i, s):
        sf = jnp.max(s.reshape(5, 8, LANES), axis=0)
        sf = jnp.max(sf, axis=0, keepdims=True)
        m_score = jnp.max(sf)
        kf = jnp.min(jnp.where(s == m_score, idx_f, big_f).reshape(
            5, 8, LANES), axis=0)
        kf = jnp.min(kf, axis=0, keepdims=True)
        m = jnp.min(kf).astype(jnp.int32)
        bx1 = x1s_ref[m]
        by1 = y1s_ref[m]
        bx2 = x2s_ref[m]
        by2 = y2s_ref[m]
        area_m = (bx2 - bx1) * (by2 - by1)
        w = jnp.maximum(jnp.minimum(bx2, x2) - jnp.maximum(bx1, x1), 0.0)
        h = jnp.maximum(jnp.minimum(by2, y2) - jnp.maximum(by1, y1), 0.0)
        inter = w * h
        iou = inter / (area_m + area - inter + 1e-12)
        decay = jnp.where(iou > IOU_THRESHOLD, 1.0 - iou, 1.0)
        others = (s != NEG_INF) & (idx != m)
        s = jnp.where(others, s * decay, s)
        s = jnp.where(idx == m, NEG_INF, s)
        order_ref[i] = m
        score_ref[i] = m_score
        return s

    jax.lax.fori_loop(0, N, body, s_ref[...])


def _global_path(obox, scores, idxs, nmax):
    del idxs, nmax
    pad = GPAD - N
    x1 = jnp.pad(obox[:, 0], (0, pad)).reshape(GROWS, LANES)
    y1 = jnp.pad(obox[:, 1], (0, pad)).reshape(GROWS, LANES)
    x2 = jnp.pad(obox[:, 2], (0, pad)).reshape(GROWS, LANES)
    y2 = jnp.pad(obox[:, 3], (0, pad)).reshape(GROWS, LANES)
    s = jnp.pad(scores, (0, pad), constant_values=NEG_INF).reshape(
        GROWS, LANES)

    vspec = pl.BlockSpec(memory_space=pltpu.VMEM)
    sspec = pl.BlockSpec(memory_space=pltpu.SMEM)
    order, sel_scores = pl.pallas_call(
        _global_kernel,
        in_specs=[vspec] * 5 + [sspec] * 4,
        out_specs=(sspec, sspec),
        out_shape=(jax.ShapeDtypeStruct((N,), jnp.int32),
                   jax.ShapeDtypeStruct((N,), jnp.float32)),
    )(x1, y1, x2, y2, s, obox[:, 0], obox[:, 1], obox[:, 2], obox[:, 3])
    return order, sel_scores


def kernel(boxes, scores, idxs):
    max_coord = jnp.max(boxes)
    offsets = idxs.astype(boxes.dtype) * (max_coord + 1.0)
    obox = boxes + offsets[:, None]

    counts = jnp.bincount(idxs, length=NUM_CLASSES)
    nmax = jnp.max(counts).astype(jnp.int32)
    order, sel_scores = jax.lax.cond(
        nmax <= CAP, _vec_path, _global_path, obox, scores, idxs, nmax)
    keep_mask = sel_scores > SCORE_THRESHOLD
    return order, sel_scores, keep_mask
